# Initial kernel scaffold; baseline (speedup 1.0000x reference)
import jax
import jax.numpy as jnp
from jax.experimental import pallas as pl


def kernel(x, edge_weight, ln1_g, ln1_b, Wl, bl, Wr, br, We, att, gat_bias, ln2_g, ln2_b, W1, b1, W2, b2, edge_index):
    raise NotImplementedError("write your pallas kernel here")


# speedup vs baseline: 9.6540x; 9.6540x over previous
import functools
import math

import jax
import jax.numpy as jnp
from jax import lax
from jax.experimental import pallas as pl
from jax.experimental.pallas import tpu as pltpu
from jax.experimental.pallas import tpu_sc as plsc

N = 10000
E = 320000
D = 128
H = 8
C = 128
HC = H * C
HP = 16
NC = 2
NS = 16
NW = NC * NS
EW = E // NW
B = 16
NBATCH = EW // B
NPAIR = NBATCH // 2
CHB = 25
CHE = CHB * B
NP = 10240
ROWS_T = NP // NS
WPT = ROWS_T * H
NRT = N // NS
DB = 80
NDB = EW // DB
ECHB = 5

_f32 = jnp.float32
_i32 = jnp.int32

_sc_mesh = plsc.VectorSubcoreMesh(core_axis_name="c", subcore_axis_name="s")


def _wid():
    return lax.axis_index("s") * NC + lax.axis_index("c")


def _pre_body(x_ref, g_ref, b_ref, wl_ref, bl_ref, wr_ref, br_ref,
              xl_ref, xr_ref):
    x = x_ref[...]
    mu = jnp.mean(x, axis=-1, keepdims=True)
    xc = x - mu
    var = jnp.mean(xc * xc, axis=-1, keepdims=True)
    y = xc * lax.rsqrt(var + 1e-5) * g_ref[...] + b_ref[...]
    xl_ref[...] = (jnp.dot(y, wl_ref[...], preferred_element_type=_f32)
                   + bl_ref[...])
    xr_ref[...] = (jnp.dot(y, wr_ref[...], preferred_element_type=_f32)
                   + br_ref[...])


def _pre_call(x, g, b, wlT, bl, wrT, br):
    blk = 1000
    return pl.pallas_call(
        _pre_body,
        grid=(N // blk,),
        in_specs=[
            pl.BlockSpec((blk, D), lambda i: (i, 0)),
            pl.BlockSpec((1, D), lambda i: (0, 0)),
            pl.BlockSpec((1, D), lambda i: (0, 0)),
            pl.BlockSpec((D, HC), lambda i: (0, 0)),
            pl.BlockSpec((1, HC), lambda i: (0, 0)),
            pl.BlockSpec((D, HC), lambda i: (0, 0)),
            pl.BlockSpec((1, HC), lambda i: (0, 0)),
        ],
        out_specs=[
            pl.BlockSpec((blk, HC), lambda i: (i, 0)),
            pl.BlockSpec((blk, HC), lambda i: (i, 0)),
        ],
        out_shape=[
            jax.ShapeDtypeStruct((N, HC), _f32),
            jax.ShapeDtypeStruct((N, HC), _f32),
        ],
    )(x, g, b, wlT, bl, wrT, br)


def _passB_body(xl_hbm, xr_hbm, src_hbm, dst_hbm, ew_hbm, we_hbm, att_hbm,
                alpha_hbm,
                src_v, dst_v, ew_v, we_v, att_v,
                rl0, rr0, rl1, rr1, a_buf,
                sl0, sr0, sl1, sr1):
    base = _wid() * EW
    pltpu.sync_copy(src_hbm.at[pl.ds(base, EW)], src_v)
    pltpu.sync_copy(dst_hbm.at[pl.ds(base, EW)], dst_v)
    pltpu.sync_copy(ew_hbm.at[pl.ds(base, EW)], ew_v.at[pl.ds(0, EW)])
    pltpu.sync_copy(we_hbm, we_v)
    pltpu.sync_copy(att_hbm, att_v)

    iota = lax.iota(_i32, 16)

    def issue(k, rl, rr, sl, sr):
        il = src_v[pl.ds(k * B, B)]
        ir = dst_v[pl.ds(k * B, B)]
        pltpu.async_copy(xl_hbm.at[il], rl, sl)
        pltpu.async_copy(xr_hbm.at[ir], rr, sr)

    def wait(k, rl, rr, sl, sr):
        il = src_v[pl.ds(k * B, B)]
        ir = dst_v[pl.ds(k * B, B)]
        pltpu.make_async_copy(xl_hbm.at[il], rl, sl).wait()
        pltpu.make_async_copy(xr_hbm.at[ir], rr, sr).wait()

    def compute(k, rl, rr):
        kc = lax.rem(k, CHB)

        def edge(e, _):
            ew_s = ew_v[pl.ds(k * B + e, 16)][0]
            row = kc * B + e
            av = jnp.zeros((16,), _f32)
            for h in range(H):
                acc = jnp.zeros((16,), _f32)
                for cc in range(C // 16):
                    off = h * C + cc * 16
                    z = (rl[e, pl.ds(off, 16)] + rr[e, pl.ds(off, 16)]
                         + ew_s * we_v[pl.ds(off, 16)])
                    z = jnp.maximum(z, 0.2 * z)
                    acc = acc + z * att_v[pl.ds(off, 16)]
                for kk in (8, 4, 2, 1):
                    acc = acc + acc[lax.bitwise_xor(iota, kk)]
                av = jnp.where(iota == h, acc, av)
            a_buf[pl.ds(row * HP, 16)] = av
            return 0
        lax.fori_loop(0, B, edge, 0)

        @pl.when(kc == CHB - 1)
        def _():
            pltpu.sync_copy(
                a_buf,
                alpha_hbm.at[pl.ds((base + (k - (CHB - 1)) * B) * HP,
                                   CHE * HP)])

    issue(0, rl0, rr0, sl0, sr0)

    def pair(j, _):
        k0 = 2 * j
        k1 = 2 * j + 1
        issue(k1, rl1, rr1, sl1, sr1)
        wait(k0, rl0, rr0, sl0, sr0)
        compute(k0, rl0, rr0)
        issue(k0 + 2, rl0, rr0, sl0, sr0)
        wait(k1, rl1, rr1, sl1, sr1)
        compute(k1, rl1, rr1)
        return 0
    lax.fori_loop(0, NPAIR, pair, 0)
    wait(NBATCH - 1, rl0, rr0, sl0, sr0)
    compute(NBATCH - 1, rl0, rr0)


def _passB(xl, xr, src, dst, ewf, wef, attf):
    f = functools.partial(
        pl.kernel,
        out_type=jax.ShapeDtypeStruct((E * HP,), _f32),
        mesh=_sc_mesh,
        scratch_types=[
            pltpu.VMEM((EW,), _i32),
            pltpu.VMEM((EW,), _i32),
            pltpu.VMEM((EW + 16,), _f32),
            pltpu.VMEM((HC,), _f32),
            pltpu.VMEM((HC,), _f32),
            pltpu.VMEM((B, HC), _f32),
            pltpu.VMEM((B, HC), _f32),
            pltpu.VMEM((B, HC), _f32),
            pltpu.VMEM((B, HC), _f32),
            pltpu.VMEM((CHE * HP,), _f32),
            pltpu.SemaphoreType.DMA,
            pltpu.SemaphoreType.DMA,
            pltpu.SemaphoreType.DMA,
            pltpu.SemaphoreType.DMA,
        ],
    )(_passB_body)
    return f(xl, xr, src, dst, ewf, wef, attf)


NPH = NP * H


def _combine(stage_hbm, out_hbm, c, s, tmp, acc, init_val, reduce_fn):
    ini = jnp.full((16,), init_val, _f32)

    def cinit(i, _):
        acc[pl.ds(i * 16, 16)] = ini
        return 0
    lax.fori_loop(0, WPT // 16, cinit, 0)

    for t in range(NS):
        pltpu.sync_copy(
            stage_hbm.at[pl.ds((t * NC + c) * NPH + s * WPT, WPT)], tmp)

        def cred(i, _):
            sl = pl.ds(i * 16, 16)
            acc[sl] = reduce_fn(acc[sl], tmp[sl])
            return 0
        lax.fori_loop(0, WPT // 16, cred, 0)

    pltpu.sync_copy(acc, out_hbm.at[pl.ds(c * NPH + s * WPT, WPT)])


def _passC_body(alpha_hbm, dst_hbm, stage_hbm, amax_hbm,
                priv, dst_v, a_buf, tmp, acc):
    c = lax.axis_index("c")
    s = lax.axis_index("s")
    base = _wid() * EW

    neg = jnp.full((16,), -3e38, _f32)

    def init(i, _):
        priv[pl.ds(i * 16, 16)] = neg
        return 0
    lax.fori_loop(0, NP * H // 16, init, 0)

    pltpu.sync_copy(dst_hbm.at[pl.ds(base, EW)], dst_v.at[pl.ds(0, EW)])

    iota = lax.iota(_i32, 16)
    h8 = lax.bitwise_and(iota, 7)
    mask_lo = iota < 8

    def chunk(kc, _):
        pltpu.sync_copy(alpha_hbm.at[pl.ds((base + kc * CHE) * HP,
                                           CHE * HP)], a_buf)

        def edge(e, _2):
            d = dst_v[pl.ds(kc * CHE + e, 16)][0]
            av = jnp.where(mask_lo, a_buf[pl.ds(e * HP, 16)], neg)
            sl = pl.ds(d * H, 16)
            priv[sl] = jnp.maximum(priv[sl], av)
            return 0
        lax.fori_loop(0, CHE, edge, 0)
        return 0
    lax.fori_loop(0, EW // CHE, chunk, 0)

    pltpu.sync_copy(priv, stage_hbm.at[pl.ds(_wid() * NPH, NPH)])
    plsc.subcore_barrier()
    _combine(stage_hbm, amax_hbm, c, s, tmp, acc, -3e38, jnp.maximum)


def _passC(alpha, dst):
    f = functools.partial(
        pl.kernel,
        out_type=(jax.ShapeDtypeStruct((NW * NPH,), _f32),
                  jax.ShapeDtypeStruct((NC * NPH,), _f32)),
        mesh=_sc_mesh,
        scratch_types=[
            pltpu.VMEM((NP * H,), _f32),
            pltpu.VMEM((EW + 16,), _i32),
            pltpu.VMEM((CHE * HP,), _f32),
            pltpu.VMEM((WPT,), _f32),
            pltpu.VMEM((WPT,), _f32),
        ],
    )(_passC_body)
    _, amax_p = f(alpha, dst)
    return amax_p


def _fin1_body(a_ref, o_ref):
    o_ref[...] = jnp.maximum(a_ref[0], a_ref[1])


def _fin1(amax_p):
    return pl.pallas_call(
        _fin1_body,
        out_shape=jax.ShapeDtypeStruct((NP * H // 128, 128), _f32),
    )(amax_p)


def _fin2_body(d_ref, o_ref):
    o_ref[...] = 1.0 / (H * (d_ref[0] + d_ref[1] + 1e-16))


def _fin2(denom_p):
    return pl.pallas_call(
        _fin2_body,
        out_shape=jax.ShapeDtypeStruct((NP * H // 128, 128), _f32),
    )(denom_p)


def _passD_body(alpha_hbm, dst_hbm, amax_hbm, ae_hbm, stage_hbm, den_hbm,
                priv, dst_v, a_buf, ae_buf, amx_buf, idx_b, tmp, acc, gsem):
    c = lax.axis_index("c")
    s = lax.axis_index("s")
    base = _wid() * EW

    zero = jnp.zeros((16,), _f32)

    def init(i, _):
        priv[pl.ds(i * 16, 16)] = zero
        return 0
    lax.fori_loop(0, NP * H // 16, init, 0)

    pltpu.sync_copy(dst_hbm.at[pl.ds(base, EW)], dst_v.at[pl.ds(0, EW)])

    iota = lax.iota(_i32, 16)
    mask_lo = iota < 8

    def batch(k, _):
        e0 = k * DB

        def sh(i, _2):
            idx_b[pl.ds(i * 16, 16)] = lax.shift_right_logical(
                dst_v[pl.ds(e0 + i * 16, 16)], 4)
            return 0
        lax.fori_loop(0, DB // 16, sh, 0)

        cp = pltpu.async_copy(amax_hbm.at[idx_b],
                              amx_buf.at[pl.ds(0, DB)], gsem)
        pltpu.sync_copy(alpha_hbm.at[pl.ds((base + e0) * HP, DB * HP)],
                        a_buf)
        cp.wait()

        def edge(e, _2):
            d = dst_v[pl.ds(e0 + e, 16)][0]
            roff = lax.bitwise_and(d, 15) * H
            amx = amx_buf[e, pl.ds(roff, 16)]
            ae = jnp.exp(jnp.where(mask_lo,
                                   a_buf[pl.ds(e * HP, 16)] - amx, -1e30))
            ae_buf[pl.ds(e * HP, 16)] = ae
            sl = pl.ds(d * H, 16)
            priv[sl] = priv[sl] + jnp.where(mask_lo, ae, 0.0)
            return 0
        lax.fori_loop(0, DB, edge, 0)

        pltpu.sync_copy(ae_buf, ae_hbm.at[pl.ds((base + e0) * HP, DB * HP)])
        return 0
    lax.fori_loop(0, NDB, batch, 0)

    pltpu.sync_copy(priv, stage_hbm.at[pl.ds(_wid() * NPH, NPH)])
    plsc.subcore_barrier()
    _combine(stage_hbm, den_hbm, c, s, tmp, acc, 0.0, lax.add)


def _passD(alpha, dst, amax_c):
    f = functools.partial(
        pl.kernel,
        out_type=(jax.ShapeDtypeStruct((E * HP,), _f32),
                  jax.ShapeDtypeStruct((NW * NPH,), _f32),
                  jax.ShapeDtypeStruct((NC * NPH,), _f32)),
        mesh=_sc_mesh,
        scratch_types=[
            pltpu.VMEM((NP * H,), _f32),
            pltpu.VMEM((EW + 16,), _i32),
            pltpu.VMEM((DB * HP,), _f32),
            pltpu.VMEM((DB * HP,), _f32),
            pltpu.VMEM((DB + 1, 128), _f32),
            pltpu.VMEM((DB,), _i32),
            pltpu.VMEM((WPT,), _f32),
            pltpu.VMEM((WPT,), _f32),
            pltpu.SemaphoreType.DMA,
        ],
    )(_passD_body)
    ae, _, den_p = f(alpha, dst, amax_c)
    return ae, den_p


def _passE_body(xl_hbm, src_hbm, dst_hbm, ae_hbm, inv_hbm, num_hbm,
                src_ch, dst_ch, rl0, rl1, iv0, iv1, ae_ch, a_sc, w_buf,
                dst_b, tb,
                sl0, sl1, si0, si1, num_sp):
    c = lax.axis_index("c")
    s = lax.axis_index("s")
    base = _wid() * EW

    def z(q, _):
        tb[q // (D // 16), pl.ds((q % (D // 16)) * 16, 16)] = (
            jnp.zeros((16,), _f32))
        return 0
    lax.fori_loop(0, 8 * D // 16, z, 0)

    def zc(r, _):
        @pl.when(lax.rem(r, NS) == s)
        def _():
            pltpu.sync_copy(tb, num_sp.at[pl.ds(r * 8, 8)])
        return 0
    lax.fori_loop(0, N // 8, zc, 0)
    plsc.subcore_barrier()

    def load_chunk(jc):
        q = lax.rem(jc, 2)
        pltpu.sync_copy(src_hbm.at[pl.ds(base + jc * CHE, CHE)],
                        src_ch.at[pl.ds(q * CHE, CHE)])
        pltpu.sync_copy(dst_hbm.at[pl.ds(base + jc * CHE, CHE)],
                        dst_ch.at[pl.ds(q * CHE, CHE)])
        return 0

    iota = lax.iota(_i32, 16)
    mask_lo = iota < 8

    def _idx(k):
        q = lax.rem(k // CHB, 2)
        r = lax.rem(k, CHB)
        return (src_ch[pl.ds(q * CHE + r * B, B)],
                lax.shift_right_logical(
                    dst_ch[pl.ds(q * CHE + r * B, B)], 4))

    def issue(k, rl, iv, sl, si):
        @pl.when(jnp.logical_and(lax.rem(k, CHB) == 0, k > 0))
        def _():
            load_chunk(k // CHB)
        isrc, idst = _idx(k)
        pltpu.async_copy(xl_hbm.at[isrc], rl, sl)
        pltpu.async_copy(inv_hbm.at[idst], iv.at[pl.ds(0, B)], si)

    def wait(k, rl, iv, sl, si):
        isrc, idst = _idx(k)
        pltpu.make_async_copy(xl_hbm.at[isrc], rl, sl).wait()
        pltpu.make_async_copy(inv_hbm.at[idst], iv.at[pl.ds(0, B)],
                              si).wait()

    def compute(k, rl, iv):
        kc = lax.rem(k, CHB)
        q = lax.rem(k // CHB, 2)

        ka = lax.rem(k, ECHB)

        @pl.when(ka == 0)
        def _():
            pltpu.sync_copy(ae_hbm.at[pl.ds((base + k * B) * HP,
                                            ECHB * B * HP)], ae_ch)

        def scale(e, _):
            d = dst_ch[pl.ds(q * CHE + kc * B + e, 16)][0]
            roff = lax.bitwise_and(d, 15) * H
            ivl = iv[e, pl.ds(roff, 16)]
            aev = ae_ch[pl.ds((ka * B + e) * HP, 16)]
            a_sc[pl.ds(e * HP, 16)] = jnp.where(mask_lo, aev * ivl, 0.0)
            return 0
        lax.fori_loop(0, B, scale, 0)

        def edge(e, _):
            avec = a_sc[pl.ds(e * HP, 16)]
            ah = [avec[h] for h in range(H)]
            for cc in range(C // 16):
                acc = ah[0] * rl[e, pl.ds(cc * 16, 16)]
                for h in range(1, H):
                    acc = acc + ah[h] * rl[e, pl.ds(h * C + cc * 16, 16)]
                w_buf[e, pl.ds(cc * 16, 16)] = acc
            return 0
        lax.fori_loop(0, B, edge, 0)

        dst_b[...] = dst_ch[pl.ds(q * CHE + kc * B, B)]
        pltpu.sync_copy(w_buf, num_sp.at[dst_b], add=True)

    load_chunk(0)
    issue(0, rl0, iv0, sl0, si0)

    def pair(j, _):
        k0 = 2 * j
        k1 = 2 * j + 1
        issue(k1, rl1, iv1, sl1, si1)
        wait(k0, rl0, iv0, sl0, si0)
        compute(k0, rl0, iv0)
        issue(k0 + 2, rl0, iv0, sl0, si0)
        wait(k1, rl1, iv1, sl1, si1)
        compute(k1, rl1, iv1)
        return 0
    lax.fori_loop(0, NPAIR, pair, 0)
    wait(NBATCH - 1, rl0, iv0, sl0, si0)
    compute(NBATCH - 1, rl0, iv0)

    plsc.subcore_barrier()

    def dump(r, _):
        @pl.when(lax.rem(r, NS) == s)
        def _():
            pltpu.sync_copy(num_sp.at[pl.ds(r * 8, 8)], tb)
            pltpu.sync_copy(tb, num_hbm.at[c].at[pl.ds(r * 8, 8)])
        return 0
    lax.fori_loop(0, N // 8, dump, 0)


def _passE(xl, src, dst, ae, inv):
    f = functools.partial(
        pl.kernel,
        out_type=jax.ShapeDtypeStruct((NC, N, D), _f32),
        mesh=_sc_mesh,
        scratch_types=[
            pltpu.VMEM((2 * CHE,), _i32),
            pltpu.VMEM((2 * CHE + 16,), _i32),
            pltpu.VMEM((B, HC), _f32),
            pltpu.VMEM((B, HC), _f32),
            pltpu.VMEM((B + 1, 128), _f32),
            pltpu.VMEM((B + 1, 128), _f32),
            pltpu.VMEM((ECHB * B * HP,), _f32),
            pltpu.VMEM((B * HP,), _f32),
            pltpu.VMEM((B, D), _f32),
            pltpu.VMEM((B,), _i32),
            pltpu.VMEM((8, D), _f32),
            pltpu.SemaphoreType.DMA,
            pltpu.SemaphoreType.DMA,
            pltpu.SemaphoreType.DMA,
            pltpu.SemaphoreType.DMA,
            pltpu.VMEM_SHARED((N, D), _f32),
        ],
    )(_passE_body)
    return f(xl, src, dst, ae, inv)


def _post_body(x_ref, n0_ref, n1_ref, gb_ref, g2_ref, b2_ref,
               w1_ref, b1_ref, w2_ref, b2b_ref, o_ref):
    x2 = x_ref[...] + n0_ref[...] + n1_ref[...] + gb_ref[...]
    mu = jnp.mean(x2, axis=-1, keepdims=True)
    xc = x2 - mu
    var = jnp.mean(xc * xc, axis=-1, keepdims=True)
    y = xc * lax.rsqrt(var + 1e-5) * g2_ref[...] + b2_ref[...]
    h1 = jnp.dot(y, w1_ref[...], preferred_element_type=_f32) + b1_ref[...]
    h1 = 0.5 * h1 * (1.0 + lax.erf(h1 * (1.0 / math.sqrt(2.0))))
    y2 = jnp.dot(h1, w2_ref[...], preferred_element_type=_f32) + b2b_ref[...]
    o_ref[...] = x2 + y2


def _post_call(x, n0, n1, gb, g2, b2, w1T, b1, w2T, b2b):
    blk = 1000
    return pl.pallas_call(
        _post_body,
        grid=(N // blk,),
        in_specs=[
            pl.BlockSpec((blk, D), lambda i: (i, 0)),
            pl.BlockSpec((blk, D), lambda i: (i, 0)),
            pl.BlockSpec((blk, D), lambda i: (i, 0)),
            pl.BlockSpec((1, D), lambda i: (0, 0)),
            pl.BlockSpec((1, D), lambda i: (0, 0)),
            pl.BlockSpec((1, D), lambda i: (0, 0)),
            pl.BlockSpec((D, D), lambda i: (0, 0)),
            pl.BlockSpec((1, D), lambda i: (0, 0)),
            pl.BlockSpec((D, D), lambda i: (0, 0)),
            pl.BlockSpec((1, D), lambda i: (0, 0)),
        ],
        out_specs=pl.BlockSpec((blk, D), lambda i: (i, 0)),
        out_shape=jax.ShapeDtypeStruct((N, D), _f32),
    )(x, n0, n1, gb, g2, b2, w1T, b1, w2T, b2b)


def kernel(x, edge_weight, ln1_g, ln1_b, Wl, bl, Wr, br, We, att, gat_bias,
           ln2_g, ln2_b, W1, b1, W2, b2, edge_index):
    src = edge_index[0]
    dst = edge_index[1]
    ewf = edge_weight[:, 0]
    wef = We[:, 0]
    attf = att.reshape(HC)

    xl, xr = _pre_call(x, ln1_g.reshape(1, D), ln1_b.reshape(1, D),
                       Wl.T, bl.reshape(1, HC), Wr.T, br.reshape(1, HC))

    alpha = _passB(xl, xr, src, dst, ewf, wef, attf)
    amax_p = _passC(alpha, dst)
    amax_c = _fin1(amax_p.reshape(NC, NPH // 128, 128))
    ae, denom_p = _passD(alpha, dst, amax_c)
    inv = _fin2(denom_p.reshape(NC, NPH // 128, 128))
    num_p = _passE(xl, src, dst, ae, inv)

    return _post_call(x, num_p[0], num_p[1], gat_bias.reshape(1, D),
                      ln2_g.reshape(1, D), ln2_b.reshape(1, D),
                      W1.T, b1.reshape(1, D), W2.T, b2.reshape(1, D))
